# Initial kernel scaffold; baseline (speedup 1.0000x reference)
import functools

import jax
import jax.numpy as jnp
from jax import lax
from jax.experimental import pallas as pl
from jax.experimental.pallas import tpu as pltpu
from jax.experimental.pallas import tpu_sc as plsc

MAP_CELLS = 400
N_BINS = MAP_CELLS * MAP_CELLS
DUMP = N_BINS
BINS_PAD = 160256
ZSLICE = BINS_PAD // 16
OSLICE = N_BINS // 16

NW = 32
CHUNK = 1024
N_PTS = 1_000_000
PW = 31744
NP = NW * PW
N_CHUNKS = PW // CHUNK

MAGIC = 12582912.0
INV_CELL = 10.0
SHIFT = 200.0


def _sc_body(xyz, out, xv, yv, zv, idxbuf, onesv, obuf, hist, sem):
    c = lax.axis_index("c")
    s = lax.axis_index("s")
    wid = c * 16 + s

    ones16 = jnp.ones((16,), jnp.float32)
    zeros16 = jnp.zeros((16,), jnp.float32)
    for i in range(8):
        onesv[pl.ds(16 * i, 16)] = ones16

    def _zero(i, carry):
        obuf[pl.ds(i * 16, 16)] = zeros16
        return carry

    lax.fori_loop(0, ZSLICE // 16, _zero, None)
    pltpu.sync_copy(obuf, hist.at[pl.ds(s * ZSLICE, ZSLICE)])
    plsc.subcore_barrier()

    base = wid * PW

    def _chunk(k, carry):
        off = base + k * CHUNK
        cx = pltpu.async_copy(xyz.at[0, pl.ds(off, CHUNK)], xv, sem)
        cy = pltpu.async_copy(xyz.at[1, pl.ds(off, CHUNK)], yv, sem)
        cz = pltpu.async_copy(xyz.at[2, pl.ds(off, CHUNK)], zv, sem)
        cx.wait()
        cy.wait()
        cz.wait()
        for r in range(8):
            for j in range(8):
                g = (r * 8 + j) * 16
                xs = xv[pl.ds(g, 16)]
                ys = yv[pl.ds(g, 16)]
                zs = zv[pl.ds(g, 16)]
                rr = (zs * INV_CELL + SHIFT + MAGIC) - MAGIC
                rc = (xs * INV_CELL + SHIFT + MAGIC) - MAGIC
                keep = (ys < 0.0) & (ys > -1.0)
                keep &= (rr >= 0.0) & (rr <= 399.0)
                keep &= (rc >= 0.0) & (rc <= 399.0)
                idxf = jnp.where(keep, rr * 400.0 + rc, float(DUMP))
                idxbuf[r, pl.ds(j * 16, 16)] = idxf.astype(jnp.int32)
        for r in range(8):
            pltpu.sync_copy(onesv, hist.at[idxbuf.at[r]], add=True)
        return carry

    lax.fori_loop(0, N_CHUNKS, _chunk, None)
    plsc.subcore_barrier()

    pltpu.sync_copy(hist.at[pl.ds(s * OSLICE, OSLICE)],
                    obuf.at[pl.ds(0, OSLICE)])
    pltpu.sync_copy(obuf.at[pl.ds(0, OSLICE)],
                    out.at[c, pl.ds(s * OSLICE, OSLICE)])


_sc_hist = pl.kernel(
    _sc_body,
    out_type=jax.ShapeDtypeStruct((2, N_BINS), jnp.float32),
    mesh=plsc.VectorSubcoreMesh(core_axis_name="c", subcore_axis_name="s"),
    scratch_types=[
        pltpu.VMEM((CHUNK,), jnp.float32),
        pltpu.VMEM((CHUNK,), jnp.float32),
        pltpu.VMEM((CHUNK,), jnp.float32),
        pltpu.VMEM((8, 128), jnp.int32),
        pltpu.VMEM((128,), jnp.float32),
        pltpu.VMEM((ZSLICE,), jnp.float32),
        pltpu.VMEM_SHARED((BINS_PAD,), jnp.float32),
        pltpu.SemaphoreType.DMA,
    ],
)


def _sum_body(p_ref, o_ref):
    o_ref[...] = p_ref[0] + p_ref[1]


def kernel(sparse_depth):
    n = sparse_depth.shape[0]
    pts = jnp.zeros((3, NP), jnp.float32).at[:, :n].set(sparse_depth.T)
    partial = _sc_hist(pts)
    out = pl.pallas_call(
        _sum_body,
        out_shape=jax.ShapeDtypeStruct((1250, 128), jnp.float32),
    )(partial.reshape(2, 1250, 128))
    return out.reshape(MAP_CELLS, MAP_CELLS)


# speedup vs baseline: 2.5871x; 2.5871x over previous
import functools

import jax
import jax.numpy as jnp
from jax import lax
from jax.experimental import pallas as pl
from jax.experimental.pallas import tpu as pltpu
from jax.experimental.pallas import tpu_sc as plsc

MAP_CELLS = 400
N_BINS = MAP_CELLS * MAP_CELLS
DUMP = N_BINS
BINS_PAD = 160256
ZSLICE = BINS_PAD // 16
OSLICE = N_BINS // 16

NW = 32
CHUNK = 1024
N_PTS = 1_000_000
PW = 31744
NP = NW * PW
N_CHUNKS = PW // CHUNK

MAGIC = 12582912.0
CELL = 0.1
SHIFT = 200.0


def _sc_body(xh, yh, zh, out, xv, yv, zv, idxbuf, onesv, obuf, hist, sem):
    c = lax.axis_index("c")
    s = lax.axis_index("s")
    wid = c * 16 + s

    ones16 = jnp.ones((16,), jnp.float32)
    zeros16 = jnp.zeros((16,), jnp.float32)
    for i in range(8):
        onesv[pl.ds(16 * i, 16)] = ones16

    def _zero(i, carry):
        obuf[pl.ds(i * 16, 16)] = zeros16
        return carry

    lax.fori_loop(0, ZSLICE // 16, _zero, None)
    pltpu.sync_copy(obuf, hist.at[pl.ds(s * ZSLICE, ZSLICE)])
    plsc.subcore_barrier()

    base = wid * PW

    def _chunk(k, carry):
        off = base + k * CHUNK
        cx = pltpu.async_copy(xh.at[pl.ds(off, CHUNK)], xv, sem)
        cy = pltpu.async_copy(yh.at[pl.ds(off, CHUNK)], yv, sem)
        cz = pltpu.async_copy(zh.at[pl.ds(off, CHUNK)], zv, sem)
        cx.wait()
        cy.wait()
        cz.wait()
        for r in range(8):
            for j in range(8):
                g = (r * 8 + j) * 16
                xs = xv[pl.ds(g, 16)]
                ys = yv[pl.ds(g, 16)]
                zs = zv[pl.ds(g, 16)]
                rr = (zs / CELL + SHIFT + MAGIC) - MAGIC
                rc = (xs / CELL + SHIFT + MAGIC) - MAGIC
                keep = (ys < 0.0) & (ys > -1.0)
                keep &= (rr >= 0.0) & (rr <= 399.0)
                keep &= (rc >= 0.0) & (rc <= 399.0)
                idxf = jnp.where(keep, rr * 400.0 + rc, float(DUMP))
                idxbuf[r, pl.ds(j * 16, 16)] = idxf.astype(jnp.int32)
        for r in range(8):
            pltpu.sync_copy(onesv, hist.at[idxbuf.at[r]], add=True)
        return carry

    lax.fori_loop(0, N_CHUNKS, _chunk, None)
    plsc.subcore_barrier()

    pltpu.sync_copy(hist.at[pl.ds(s * OSLICE, OSLICE)],
                    obuf.at[pl.ds(0, OSLICE)])
    pltpu.sync_copy(obuf.at[pl.ds(0, OSLICE)],
                    out.at[pl.ds(c * N_BINS + s * OSLICE, OSLICE)])


_sc_hist = pl.kernel(
    _sc_body,
    out_type=jax.ShapeDtypeStruct((2 * N_BINS,), jnp.float32),
    mesh=plsc.VectorSubcoreMesh(core_axis_name="c", subcore_axis_name="s"),
    scratch_types=[
        pltpu.VMEM((CHUNK,), jnp.float32),
        pltpu.VMEM((CHUNK,), jnp.float32),
        pltpu.VMEM((CHUNK,), jnp.float32),
        pltpu.VMEM((8, 128), jnp.int32),
        pltpu.VMEM((128,), jnp.float32),
        pltpu.VMEM((ZSLICE,), jnp.float32),
        pltpu.VMEM_SHARED((BINS_PAD,), jnp.float32),
        pltpu.SemaphoreType.DMA,
    ],
)


def _sum_body(p_ref, o_ref):
    o_ref[...] = p_ref[0] + p_ref[1]


def kernel(sparse_depth):
    n = sparse_depth.shape[0]
    pts = jnp.pad(sparse_depth.T, ((0, 0), (0, NP - n)))
    partial = _sc_hist(pts[0], pts[1], pts[2])
    out = pl.pallas_call(
        _sum_body,
        out_shape=jax.ShapeDtypeStruct((1250, 128), jnp.float32),
    )(partial.reshape(2, 1250, 128))
    return out.reshape(MAP_CELLS, MAP_CELLS)


# speedup vs baseline: 9.9537x; 3.8474x over previous
import jax
import jax.numpy as jnp
from jax import lax
from jax.experimental import pallas as pl
from jax.experimental.pallas import tpu as pltpu
from jax.experimental.pallas import tpu_sc as plsc

MAP_CELLS = 400
N_BINS = MAP_CELLS * MAP_CELLS
DUMP = N_BINS
BINS_PAD = 160256
ZSLICE = BINS_PAD // 16
OSLICE = N_BINS // 16

WR0 = 120
WRN = 160
WC0 = 128
WCN = 144
WSZ = WRN * WCN
MROWS = WRN // 16

NW = 32
CHUNK = 2048
GROUPS = CHUNK // 16
N_CHUNKS = 16
PW = CHUNK * N_CHUNKS
NP = NW * PW

MAGIC = 12582912.0
CELL = 0.1
SHIFT = 200.0


def _sc_body(xh, yh, zh, out, xv0, yv0, zv0, xv1, yv1, zv1, idxbuf,
             onesv, wbuf, obuf, mbuf, tbuf, hist, stag, sem0, sem1):
    c = lax.axis_index("c")
    s = lax.axis_index("s")
    base = (c * 16 + s) * PW

    ones16 = jnp.ones((16,), jnp.float32)
    zeros16 = jnp.zeros((16,), jnp.float32)
    for i in range(8):
        onesv[pl.ds(16 * i, 16)] = ones16

    def _zero(i, carry):
        wbuf[pl.ds(i * 16, 16)] = zeros16
        return carry

    lax.fori_loop(0, WSZ // 16, _zero, None)
    pltpu.sync_copy(wbuf.at[pl.ds(0, ZSLICE)], hist.at[pl.ds(s * ZSLICE, ZSLICE)])
    plsc.subcore_barrier()

    def _compute(k, bufs, sem):
        xv, yv, zv = bufs
        off = base + k * CHUNK
        pltpu.make_async_copy(xh.at[pl.ds(off, CHUNK)], xv, sem).wait()
        pltpu.make_async_copy(yh.at[pl.ds(off, CHUNK)], yv, sem).wait()
        pltpu.make_async_copy(zh.at[pl.ds(off, CHUNK)], zv, sem).wait()

        rany = jnp.zeros((16,), jnp.bool_)
        for r in range(GROUPS // 8):
            def _g8(j, ra, r=r):
                o = r * 128 + j * 16
                xs = xv[pl.ds(o, 16)]
                ys = yv[pl.ds(o, 16)]
                zs = zv[pl.ds(o, 16)]
                rr = (zs / CELL + SHIFT + MAGIC) - MAGIC
                rc = (xs / CELL + SHIFT + MAGIC) - MAGIC
                masky = (ys < 0.0) & (ys > -1.0)
                inwin = ((rr >= float(WR0)) & (rr < float(WR0 + WRN))
                         & (rc >= float(WC0)) & (rc < float(WC0 + WCN))
                         & masky)
                wif = rr * float(WCN) + rc - float(WR0 * WCN + WC0)
                widx = jnp.where(inwin, wif, 0.0).astype(jnp.int32)
                plsc.addupdate_scatter(wbuf, [widx], ones16, mask=inwin)
                inb = (rr >= 0.0) & (rr <= 399.0) & (rc >= 0.0) & (rc <= 399.0)
                rest = masky & inb & (~inwin)
                idxf = jnp.where(rest, rr * 400.0 + rc, float(DUMP))
                idxbuf[r, pl.ds(j * 16, 16)] = idxf.astype(jnp.int32)
                return ra | rest

            rany = lax.fori_loop(0, 8, _g8, rany)
        fire = jnp.any(rany)

        @pl.when(fire)
        def _slow():
            for rr_ in range(GROUPS // 8):
                pltpu.sync_copy(onesv, hist.at[idxbuf.at[rr_]], add=True)

    def _fire(k, bufs, sem):
        xv, yv, zv = bufs
        off = base + k * CHUNK
        pltpu.async_copy(xh.at[pl.ds(off, CHUNK)], xv, sem)
        pltpu.async_copy(yh.at[pl.ds(off, CHUNK)], yv, sem)
        pltpu.async_copy(zh.at[pl.ds(off, CHUNK)], zv, sem)

    bufs0 = (xv0, yv0, zv0)
    bufs1 = (xv1, yv1, zv1)
    _fire(0, bufs0, sem0)
    _fire(1, bufs1, sem1)

    def _pair(kk, carry):
        k0 = kk * 2
        _compute(k0, bufs0, sem0)

        @pl.when(k0 + 2 < N_CHUNKS)
        def _f0():
            _fire(k0 + 2, bufs0, sem0)

        _compute(k0 + 1, bufs1, sem1)

        @pl.when(k0 + 3 < N_CHUNKS)
        def _f1():
            _fire(k0 + 3, bufs1, sem1)

        return carry

    lax.fori_loop(0, N_CHUNKS // 2, _pair, None)
    plsc.subcore_barrier()

    pltpu.sync_copy(wbuf, stag.at[pl.ds(s * WSZ, WSZ)])
    plsc.subcore_barrier()

    woff = s * MROWS * WCN

    def _acc(i, carry):
        o = i * 16
        mbuf[pl.ds(o, 16)] = mbuf[pl.ds(o, 16)] + tbuf[pl.ds(o, 16)]
        return carry

    pltpu.sync_copy(stag.at[pl.ds(woff, MROWS * WCN)], mbuf)
    for p in range(1, 16):
        pltpu.sync_copy(stag.at[pl.ds(p * WSZ + woff, MROWS * WCN)], tbuf)
        lax.fori_loop(0, MROWS * WCN // 16, _acc, None)
    for rl in range(MROWS):
        gr = WR0 + s * MROWS + rl
        pltpu.sync_copy(mbuf.at[pl.ds(rl * WCN, WCN)],
                        hist.at[pl.ds(gr * 400 + WC0, WCN)])
    plsc.subcore_barrier()

    pltpu.sync_copy(hist.at[pl.ds(s * OSLICE, OSLICE)],
                    obuf.at[pl.ds(0, OSLICE)])
    pltpu.sync_copy(obuf.at[pl.ds(0, OSLICE)],
                    out.at[pl.ds(c * N_BINS + s * OSLICE, OSLICE)])


_sc_hist = pl.kernel(
    _sc_body,
    out_type=jax.ShapeDtypeStruct((2 * N_BINS,), jnp.float32),
    mesh=plsc.VectorSubcoreMesh(core_axis_name="c", subcore_axis_name="s"),
    compiler_params=pltpu.CompilerParams(needs_layout_passes=False),
    scratch_types=[
        pltpu.VMEM((CHUNK,), jnp.float32),
        pltpu.VMEM((CHUNK,), jnp.float32),
        pltpu.VMEM((CHUNK,), jnp.float32),
        pltpu.VMEM((CHUNK,), jnp.float32),
        pltpu.VMEM((CHUNK,), jnp.float32),
        pltpu.VMEM((CHUNK,), jnp.float32),
        pltpu.VMEM((GROUPS // 8, 128), jnp.int32),
        pltpu.VMEM((128,), jnp.float32),
        pltpu.VMEM((WSZ,), jnp.float32),
        pltpu.VMEM((ZSLICE,), jnp.float32),
        pltpu.VMEM((MROWS * WCN,), jnp.float32),
        pltpu.VMEM((MROWS * WCN,), jnp.float32),
        pltpu.VMEM_SHARED((BINS_PAD,), jnp.float32),
        pltpu.VMEM_SHARED((16 * WSZ,), jnp.float32),
        pltpu.SemaphoreType.DMA,
        pltpu.SemaphoreType.DMA,
    ],
)


def _sum_body(p_ref, o_ref):
    o_ref[...] = p_ref[0] + p_ref[1]


def kernel(sparse_depth):
    n = sparse_depth.shape[0]
    pts = jnp.pad(sparse_depth.T, ((0, 0), (0, NP - n)))
    partial = _sc_hist(pts[0], pts[1], pts[2])
    out = pl.pallas_call(
        _sum_body,
        out_shape=jax.ShapeDtypeStruct((1250, 128), jnp.float32),
    )(partial.reshape(2, 1250, 128))
    return out.reshape(MAP_CELLS, MAP_CELLS)
